# baseline (speedup 1.0000x reference)
import functools

import jax
import jax.numpy as jnp
from jax import lax
from jax.experimental import pallas as pl
from jax.experimental.pallas import tpu as pltpu
from jax.experimental.pallas import tpu_sc as plsc

_EMBED = 1024
_SEQ = 8192
_BATCH = 4
_NW = 32
_S_PER_W = _SEQ // _NW
_C = 32
_CHUNK = _C * _EMBED
_LANES = 16
_UNROLL = 8


def _sc_body(scale_hbm, x_hbm, pos_hbm, out_hbm, sbuf, pbuf, xbuf):
    wid = lax.axis_index("s") * 2 + lax.axis_index("c")
    pltpu.sync_copy(scale_hbm, sbuf)
    sv = sbuf[...]
    s_base = wid * _S_PER_W

    def add_chunk(_):
        def body(k, carry):
            base = k * (_LANES * _UNROLL)
            for u in range(_UNROLL):
                sl = pl.ds(base + u * _LANES, _LANES)
                xbuf[sl] = xbuf[sl] + pbuf[sl] * sv
            return carry
        lax.fori_loop(0, _CHUNK // (_LANES * _UNROLL), body, 0)

    for i in range(_S_PER_W // _C):
        s0 = (s_base + i * _C) * _EMBED
        pltpu.sync_copy(pos_hbm.at[pl.ds(s0, _CHUNK)], pbuf)
        for b in range(_BATCH):
            off = b * (_SEQ * _EMBED) + s0
            pltpu.sync_copy(x_hbm.at[pl.ds(off, _CHUNK)], xbuf)
            add_chunk(None)
            pltpu.sync_copy(xbuf, out_hbm.at[pl.ds(off, _CHUNK)])


def kernel(x, pos_table, use_pos_embed):
    batch, seq_len, embed_dim = x.shape
    scale16 = jnp.full((_LANES,), jnp.asarray(use_pos_embed, jnp.float32))
    x1d = x.reshape(batch * seq_len * embed_dim)
    pos1d = pos_table[:seq_len].reshape(seq_len * embed_dim)

    mesh = plsc.VectorSubcoreMesh(core_axis_name="c", subcore_axis_name="s")
    k = functools.partial(
        pl.kernel,
        mesh=mesh,
        out_type=jax.ShapeDtypeStruct(x1d.shape, x.dtype),
        scratch_types=[
            pltpu.VMEM((_LANES,), jnp.float32),
            pltpu.VMEM((_CHUNK,), jnp.float32),
            pltpu.VMEM((_CHUNK,), jnp.float32),
        ],
    )(_sc_body)
    out = k(scale16, x1d, pos1d)
    return out.reshape(x.shape)


# speedup vs baseline: 2.1444x; 2.1444x over previous
import functools

import jax
import jax.numpy as jnp
from jax import lax
from jax.experimental import pallas as pl
from jax.experimental.pallas import tpu as pltpu
from jax.experimental.pallas import tpu_sc as plsc

_EMBED = 1024
_SEQ = 8192
_BATCH = 4
_NW = 32
_S_PER_W = _SEQ // _NW
_C = 32
_LANES = 16


def _sc_body(scale_hbm, x_hbm, pos_hbm, out_hbm, sbuf, pbuf, xbuf):
    wid = lax.axis_index("s") * 2 + lax.axis_index("c")
    pltpu.sync_copy(scale_hbm, sbuf)
    sv = sbuf[...]
    s_base = wid * _S_PER_W

    def add_chunk(_):
        def body(k, carry):
            r = k // 8
            cc = (k % 8) * 128
            for u in range(8):
                sl = (r, pl.ds(cc + u * _LANES, _LANES))
                xbuf[sl] = xbuf[sl] + pbuf[sl] * sv
            return carry
        lax.fori_loop(0, _C * 8, body, 0)

    for i in range(_S_PER_W // _C):
        s0 = s_base + i * _C
        pltpu.sync_copy(pos_hbm.at[pl.ds(s0, _C), :], pbuf)
        for b in range(_BATCH):
            pltpu.sync_copy(x_hbm.at[b, pl.ds(s0, _C), :], xbuf)
            add_chunk(None)
            pltpu.sync_copy(xbuf, out_hbm.at[b, pl.ds(s0, _C), :])


def kernel(x, pos_table, use_pos_embed):
    batch, seq_len, embed_dim = x.shape
    scale16 = jnp.full((_LANES,), jnp.asarray(use_pos_embed, jnp.float32))

    mesh = plsc.VectorSubcoreMesh(core_axis_name="c", subcore_axis_name="s")
    k = functools.partial(
        pl.kernel,
        mesh=mesh,
        out_type=jax.ShapeDtypeStruct(x.shape, x.dtype),
        scratch_types=[
            pltpu.VMEM((_LANES,), jnp.float32),
            pltpu.VMEM((_C, _EMBED), jnp.float32),
            pltpu.VMEM((_C, _EMBED), jnp.float32),
        ],
        compiler_params=pltpu.CompilerParams(use_tc_tiling_on_sc=True),
    )(_sc_body)
    return k(scale16, x, pos_table[:seq_len])


# speedup vs baseline: 3.4709x; 1.6185x over previous
import functools

import jax
import jax.numpy as jnp
from jax import lax
from jax.experimental import pallas as pl
from jax.experimental.pallas import tpu as pltpu
from jax.experimental.pallas import tpu_sc as plsc

_EMBED = 1024
_SEQ = 8192
_BATCH = 4
_NW = 32
_S_PER_W = _SEQ // _NW
_C = 16
_NCH = _S_PER_W // _C
_LANES = 16
_NBUF = 3


def _sc_body(scale_hbm, x_hbm, pos_hbm, out_hbm, sbuf,
             pb0, pb1, xb0, xb1, xb2,
             spos0, spos1, sin0, sin1, sin2, sout0, sout1, sout2):
    wid = lax.axis_index("s") * 2 + lax.axis_index("c")
    pltpu.sync_copy(scale_hbm, sbuf)
    sv = sbuf[...]
    s_base = wid * _S_PER_W

    pbufs, sposs = (pb0, pb1), (spos0, spos1)
    xbufs = (xb0, xb1, xb2)
    sins = (sin0, sin1, sin2)
    souts = (sout0, sout1, sout2)

    def pos_cp(i):
        return pltpu.make_async_copy(
            pos_hbm.at[pl.ds(s_base + i * _C, _C), :], pbufs[i % 2], sposs[i % 2])

    def in_cp(t):
        i, b = steps[t]
        return pltpu.make_async_copy(
            x_hbm.at[b, pl.ds(s_base + i * _C, _C), :], xbufs[t % _NBUF],
            sins[t % _NBUF])

    def out_cp(t):
        i, b = steps[t]
        return pltpu.make_async_copy(
            xbufs[t % _NBUF], out_hbm.at[b, pl.ds(s_base + i * _C, _C), :],
            souts[t % _NBUF])

    def add_chunk(xbuf, pbuf):
        def body(k, carry):
            r = k // 8
            cc = (k % 8) * 128
            for u in range(8):
                sl = (r, pl.ds(cc + u * _LANES, _LANES))
                xbuf[sl] = xbuf[sl] + pbuf[sl] * sv
            return carry
        lax.fori_loop(0, _C * 8, body, 0)

    steps = [(i, b) for i in range(_NCH) for b in range(_BATCH)]
    n = len(steps)

    pos_cp(0).start()
    pos_cp(1).start()
    in_cp(0).start()
    in_cp(1).start()

    for t in range(n):
        i, b = steps[t]
        if b == 0:
            pos_cp(i).wait()
        in_cp(t).wait()
        add_chunk(xbufs[t % _NBUF], pbufs[i % 2])
        out_cp(t).start()
        if b == _BATCH - 1 and i + 2 < _NCH:
            pos_cp(i + 2).start()
        if t + 2 < n:
            if t >= 1:
                out_cp(t - 1).wait()
            in_cp(t + 2).start()

    for t in range(n - _NBUF, n):
        out_cp(t).wait()


def kernel(x, pos_table, use_pos_embed):
    batch, seq_len, embed_dim = x.shape
    scale16 = jnp.full((_LANES,), jnp.asarray(use_pos_embed, jnp.float32))

    mesh = plsc.VectorSubcoreMesh(core_axis_name="c", subcore_axis_name="s")
    k = functools.partial(
        pl.kernel,
        mesh=mesh,
        out_type=jax.ShapeDtypeStruct(x.shape, x.dtype),
        scratch_types=[
            pltpu.VMEM((_LANES,), jnp.float32),
            pltpu.VMEM((_C, _EMBED), jnp.float32),
            pltpu.VMEM((_C, _EMBED), jnp.float32),
            pltpu.VMEM((_C, _EMBED), jnp.float32),
            pltpu.VMEM((_C, _EMBED), jnp.float32),
            pltpu.VMEM((_C, _EMBED), jnp.float32),
            pltpu.SemaphoreType.DMA,
            pltpu.SemaphoreType.DMA,
            pltpu.SemaphoreType.DMA,
            pltpu.SemaphoreType.DMA,
            pltpu.SemaphoreType.DMA,
            pltpu.SemaphoreType.DMA,
            pltpu.SemaphoreType.DMA,
            pltpu.SemaphoreType.DMA,
        ],
        compiler_params=pltpu.CompilerParams(use_tc_tiling_on_sc=True),
    )(_sc_body)
    return k(scale16, x, pos_table[:seq_len])
